# baseline (speedup 1.0000x reference)
import functools

import jax
import jax.numpy as jnp
from jax import lax
from jax.experimental import pallas as pl
from jax.experimental.pallas import tpu as pltpu
from jax.experimental.pallas import tpu_sc as plsc

EMB = 128
NC, NS = 2, 16
NW = NC * NS
C = 128
NBUF = 5


def _fused_body(re_ref, se_ref, out_ref):
    out_ref[...] = re_ref[...][:, None, :] + se_ref[...][None, :, :]


def _make_sc_gather(n):
    assert n % (NW * C) == 0
    n_per_w = n // NW
    nch = n_per_w // C
    assert nch % NBUF == 0 and nch >= 3 * NBUF

    mesh = plsc.VectorSubcoreMesh(
        core_axis_name="c", subcore_axis_name="s",
        num_cores=NC, num_subcores=NS)

    @functools.partial(
        pl.kernel,
        out_type=jax.ShapeDtypeStruct((n, EMB), jnp.float32),
        mesh=mesh,
        scratch_types=[
            pltpu.VMEM((NBUF, C), jnp.int32),
            pltpu.VMEM((NBUF, C), jnp.int32),
            pltpu.VMEM((NBUF, C), jnp.int32),
            pltpu.VMEM((NBUF, C, EMB), jnp.float32),
            pltpu.VMEM_SHARED((75, EMB), jnp.float32),
            pltpu.SemaphoreType.DMA((NBUF,)),
            pltpu.SemaphoreType.DMA((NBUF,)),
            pltpu.SemaphoreType.DMA((NBUF,)),
        ],
    )
    def sc_gather(rank_h, suit_h, fused_h, out_h,
                  rankv, suitv, combv, rowsv, sharedv, isem, gsem, ssem):
        sid = lax.axis_index("s")
        wid = sid * NC + lax.axis_index("c")
        base = wid * n_per_w

        @pl.when(sid == 0)
        def _stage():
            pltpu.sync_copy(fused_h, sharedv)
        plsc.subcore_barrier()

        def fire_idx(ch, b):
            off = base + ch * C
            pltpu.async_copy(rank_h.at[pl.ds(off, C)], rankv.at[b], isem.at[b])
            pltpu.async_copy(suit_h.at[pl.ds(off, C)], suitv.at[b], isem.at[b])

        def do_comb(b):
            pltpu.make_async_copy(
                rank_h.at[pl.ds(base, C)], rankv.at[b], isem.at[b]).wait()
            pltpu.make_async_copy(
                suit_h.at[pl.ds(base, C)], suitv.at[b], isem.at[b]).wait()
            for t in range(C // 16):
                sl = pl.ds(t * 16, 16)
                combv[b, sl] = rankv[b, sl] * 5 + suitv[b, sl] - 1

        def fire_gather(b, wait_scatter):
            if wait_scatter:
                pltpu.make_async_copy(
                    rowsv.at[b], out_h.at[pl.ds(base, C)], ssem.at[b]).wait()
            pltpu.async_copy(fused_h.at[combv.at[b]], rowsv.at[b], gsem.at[b])

        def do_out(ch, b):
            pltpu.make_async_copy(
                fused_h.at[combv.at[b]], rowsv.at[b], gsem.at[b]).wait()
            off = base + ch * C
            pltpu.async_copy(rowsv.at[b], out_h.at[pl.ds(off, C)], ssem.at[b])

        for ch in range(NBUF):
            fire_idx(ch, ch)
        for ch in range(3):
            do_comb(ch)
        for ch in range(2):
            fire_gather(ch, False)
        for i in range(NBUF):
            do_out(i, i)
            fire_idx(i + NBUF, i)
            do_comb((i + 3) % NBUF)
            fire_gather((i + 2) % NBUF, i + 2 >= NBUF)

        @pl.loop(NBUF, nch - NBUF, step=NBUF)
        def _steady(i0):
            for k in range(NBUF):
                i = i0 + k
                do_out(i, k)
                fire_idx(i + NBUF, k)
                do_comb((k + 3) % NBUF)
                fire_gather((k + 2) % NBUF, True)

        nb = nch - NBUF
        for i in range(nb, nch):
            do_out(i, i % NBUF)
            if i + 3 < nch:
                do_comb((i + 3) % NBUF)
            if i + 2 < nch:
                fire_gather((i + 2) % NBUF, True)
        for b in range(NBUF):
            pltpu.make_async_copy(
                rowsv.at[b], out_h.at[pl.ds(base, C)], ssem.at[b]).wait()

    return sc_gather


def kernel(rank, suit, rank_emb, suit_emb):
    bb, ll = rank.shape
    n = bb * ll
    rank_f = rank.reshape(n).astype(jnp.int32)
    suit_f = suit.reshape(n).astype(jnp.int32)
    fused3 = pl.pallas_call(
        _fused_body,
        out_shape=jax.ShapeDtypeStruct((15, 5, EMB), jnp.float32),
    )(rank_emb, suit_emb)
    fused = fused3.reshape(75, EMB)
    out = _make_sc_gather(n)(rank_f, suit_f, fused)
    return out.reshape(bb, ll, EMB)


# speedup vs baseline: 9.1714x; 9.1714x over previous
import functools

import jax
import jax.numpy as jnp
from jax import lax
from jax.experimental import pallas as pl
from jax.experimental.pallas import tpu as pltpu
from jax.experimental.pallas import tpu_sc as plsc

EMB = 128
NC, NS = 2, 16
NW = NC * NS
C = 128
NBUF = 5


def _fused_body(re_ref, se_ref, out_ref):
    out_ref[...] = re_ref[...][:, None, :] + se_ref[...][None, :, :]


def _make_sc_gather(n):
    assert n % (NW * C) == 0
    n_per_w = n // NW
    nch = n_per_w // C
    assert nch % NBUF == 0 and nch >= 3 * NBUF

    mesh = plsc.VectorSubcoreMesh(
        core_axis_name="c", subcore_axis_name="s",
        num_cores=NC, num_subcores=NS)

    @functools.partial(
        pl.kernel,
        out_type=jax.ShapeDtypeStruct((n, EMB), jnp.float32),
        mesh=mesh,
        scratch_types=[
            pltpu.VMEM((NBUF, C), jnp.int32),
            pltpu.VMEM((NBUF, C), jnp.int32),
            pltpu.VMEM((NBUF, C), jnp.int32),
            pltpu.VMEM((NBUF, C, EMB), jnp.float32),
            pltpu.VMEM_SHARED((75, EMB), jnp.float32),
            pltpu.SemaphoreType.DMA((NBUF,)),
            pltpu.SemaphoreType.DMA((NBUF,)),
            pltpu.SemaphoreType.DMA((NBUF,)),
        ],
    )
    def sc_gather(rank_h, suit_h, fused_h, out_h,
                  rankv, suitv, combv, rowsv, sharedv, isem, gsem, ssem):
        sid = lax.axis_index("s")
        wid = sid * NC + lax.axis_index("c")
        base = wid * n_per_w

        @pl.when(sid == 0)
        def _stage():
            pltpu.sync_copy(fused_h, sharedv)
        plsc.subcore_barrier()

        def fire_idx(ch, b):
            off = base + ch * C
            pltpu.async_copy(rank_h.at[pl.ds(off, C)], rankv.at[b], isem.at[b])
            pltpu.async_copy(suit_h.at[pl.ds(off, C)], suitv.at[b], isem.at[b])

        def do_comb(b):
            pltpu.make_async_copy(
                rank_h.at[pl.ds(base, C)], rankv.at[b], isem.at[b]).wait()
            pltpu.make_async_copy(
                suit_h.at[pl.ds(base, C)], suitv.at[b], isem.at[b]).wait()
            for t in range(C // 16):
                sl = pl.ds(t * 16, 16)
                combv[b, sl] = rankv[b, sl] * 5 + suitv[b, sl] - 1

        def fire_gather(b, wait_scatter):
            if wait_scatter:
                pltpu.make_async_copy(
                    rowsv.at[b], out_h.at[pl.ds(base, C)], ssem.at[b]).wait()
            pltpu.async_copy(sharedv.at[combv.at[b]], rowsv.at[b], gsem.at[b])

        def do_out(ch, b):
            pltpu.make_async_copy(
                sharedv.at[combv.at[b]], rowsv.at[b], gsem.at[b]).wait()
            off = base + ch * C
            pltpu.async_copy(rowsv.at[b], out_h.at[pl.ds(off, C)], ssem.at[b])

        for ch in range(NBUF):
            fire_idx(ch, ch)
        for ch in range(3):
            do_comb(ch)
        for ch in range(2):
            fire_gather(ch, False)
        for i in range(NBUF):
            do_out(i, i)
            fire_idx(i + NBUF, i)
            do_comb((i + 3) % NBUF)
            fire_gather((i + 2) % NBUF, i + 2 >= NBUF)

        @pl.loop(NBUF, nch - NBUF, step=NBUF)
        def _steady(i0):
            for k in range(NBUF):
                i = i0 + k
                do_out(i, k)
                fire_idx(i + NBUF, k)
                do_comb((k + 3) % NBUF)
                fire_gather((k + 2) % NBUF, True)

        nb = nch - NBUF
        for i in range(nb, nch):
            do_out(i, i % NBUF)
            if i + 3 < nch:
                do_comb((i + 3) % NBUF)
            if i + 2 < nch:
                fire_gather((i + 2) % NBUF, True)
        for b in range(NBUF):
            pltpu.make_async_copy(
                rowsv.at[b], out_h.at[pl.ds(base, C)], ssem.at[b]).wait()

    return sc_gather


def kernel(rank, suit, rank_emb, suit_emb):
    bb, ll = rank.shape
    n = bb * ll
    rank_f = rank.reshape(n).astype(jnp.int32)
    suit_f = suit.reshape(n).astype(jnp.int32)
    fused3 = pl.pallas_call(
        _fused_body,
        out_shape=jax.ShapeDtypeStruct((15, 5, EMB), jnp.float32),
    )(rank_emb, suit_emb)
    fused = fused3.reshape(75, EMB)
    out = _make_sc_gather(n)(rank_f, suit_f, fused)
    return out.reshape(bb, ll, EMB)
